# baseline (speedup 1.0000x reference)
import functools

import jax
import jax.numpy as jnp
from jax import lax
from jax.experimental import pallas as pl
from jax.experimental.pallas import tpu as pltpu

_B = 10000
_K = 16
_SIZE = 128
_BLOCK_B = 2000

_DN = (((1,), (1,)), ((), ()))


def _matmul_nt(x, w):
    return lax.dot_general(x, w, _DN, preferred_element_type=jnp.float32)


def _tree_lstm_body(ch_ref, ch0_ref, tr_ref, wiou_ref, biou_ref, wtr_ref,
                    wf_ref, bf_ref, wft_ref, out_ref, fcf_ref):
    step = pl.program_id(0)

    @pl.when(step == 0)
    def _compute_fc_first():
        c0 = ch0_ref[0]
        c0h = c0[:, :_SIZE]
        c0c = c0[:, _SIZE:]
        th0 = tr_ref[0:1, :]
        f0 = _matmul_nt(c0h, wf_ref[...]) + bf_ref[...] + _matmul_nt(th0, wft_ref[...])
        fc0 = jax.nn.sigmoid(f0) * c0c
        fcf_ref[0:1, :] = jnp.sum(fc0, axis=0, keepdims=True)

    mean_h = jnp.sum(ch_ref[...], axis=1) * (1.0 / _K)
    iou = (_matmul_nt(mean_h, wiou_ref[...]) + biou_ref[...]
           + _matmul_nt(tr_ref[...], wtr_ref[...]))
    i = jax.nn.sigmoid(iou[:, :_SIZE])
    o = jax.nn.sigmoid(iou[:, _SIZE:2 * _SIZE])
    u = jnp.tanh(iou[:, 2 * _SIZE:])
    c = i * u + fcf_ref[0:1, :]
    out_ref[:, :_SIZE] = o * c
    out_ref[:, _SIZE:] = c


@jax.jit
def _tree_lstm(childrens, tracking, W_iou, b_iou, W_forget, b_forget,
               W_iou_track, W_forget_track):
    b = childrens.shape[0]
    grid = b // _BLOCK_B
    full = lambda shape: pl.BlockSpec(shape, lambda i: (0,) * len(shape))
    return pl.pallas_call(
        _tree_lstm_body,
        grid=(grid,),
        in_specs=[
            pl.BlockSpec((_BLOCK_B, _K, _SIZE), lambda i: (i, 0, 0)),
            pl.BlockSpec((1, _K, 2 * _SIZE), lambda i: (0, 0, 0)),
            pl.BlockSpec((_BLOCK_B, _SIZE), lambda i: (i, 0)),
            full((3 * _SIZE, _SIZE)),
            full((1, 3 * _SIZE)),
            full((3 * _SIZE, _SIZE)),
            full((_SIZE, _SIZE)),
            full((1, _SIZE)),
            full((_SIZE, _SIZE)),
        ],
        out_specs=pl.BlockSpec((_BLOCK_B, 2 * _SIZE), lambda i: (i, 0)),
        out_shape=jax.ShapeDtypeStruct((b, 2 * _SIZE), jnp.float32),
        scratch_shapes=[pltpu.VMEM((8, _SIZE), jnp.float32)],
        compiler_params=pltpu.CompilerParams(
            dimension_semantics=("arbitrary",),
        ),
    )(childrens, childrens, tracking, W_iou, b_iou.reshape(1, -1),
      W_iou_track, W_forget, b_forget.reshape(1, -1), W_forget_track)


def kernel(childrens, tracking, W_iou, b_iou, W_forget, b_forget,
           W_iou_track, W_forget_track):
    return _tree_lstm(childrens, tracking, W_iou, b_iou, W_forget, b_forget,
                      W_iou_track, W_forget_track)


# speedup vs baseline: 1.0574x; 1.0574x over previous
import functools

import jax
import jax.numpy as jnp
from jax import lax
from jax.experimental import pallas as pl
from jax.experimental.pallas import tpu as pltpu

_B = 10000
_K = 16
_SIZE = 128
_BLOCK_B = 1000

_DN = (((1,), (1,)), ((), ()))


def _matmul_nt(x, w):
    return lax.dot_general(x, w, _DN, preferred_element_type=jnp.float32)


def _tree_lstm_body(ch_ref, ch0_ref, tr_ref, wiou_ref, biou_ref, wtr_ref,
                    wf_ref, bf_ref, wft_ref, out_ref, fcf_ref):
    step = pl.program_id(0)

    @pl.when(step == 0)
    def _compute_fc_first():
        c0 = ch0_ref[0]
        c0h = c0[:, :_SIZE]
        c0c = c0[:, _SIZE:]
        th0 = tr_ref[0:1, :]
        f0 = _matmul_nt(c0h, wf_ref[...]) + bf_ref[...] + _matmul_nt(th0, wft_ref[...])
        fc0 = jax.nn.sigmoid(f0) * c0c
        fcf_ref[0:1, :] = jnp.sum(fc0, axis=0, keepdims=True)

    mean_h = jnp.sum(ch_ref[...], axis=1) * (1.0 / _K)
    iou = (_matmul_nt(mean_h, wiou_ref[...]) + biou_ref[...]
           + _matmul_nt(tr_ref[...], wtr_ref[...]))
    i = jax.nn.sigmoid(iou[:, :_SIZE])
    o = jax.nn.sigmoid(iou[:, _SIZE:2 * _SIZE])
    u = jnp.tanh(iou[:, 2 * _SIZE:])
    c = i * u + fcf_ref[0:1, :]
    out_ref[:, :_SIZE] = o * c
    out_ref[:, _SIZE:] = c


@jax.jit
def _tree_lstm(childrens, tracking, W_iou, b_iou, W_forget, b_forget,
               W_iou_track, W_forget_track):
    b = childrens.shape[0]
    grid = b // _BLOCK_B
    full = lambda shape: pl.BlockSpec(shape, lambda i: (0,) * len(shape))
    return pl.pallas_call(
        _tree_lstm_body,
        grid=(grid,),
        in_specs=[
            pl.BlockSpec((_BLOCK_B, _K, _SIZE), lambda i: (i, 0, 0)),
            pl.BlockSpec((1, _K, 2 * _SIZE), lambda i: (0, 0, 0)),
            pl.BlockSpec((_BLOCK_B, _SIZE), lambda i: (i, 0)),
            full((3 * _SIZE, _SIZE)),
            full((1, 3 * _SIZE)),
            full((3 * _SIZE, _SIZE)),
            full((_SIZE, _SIZE)),
            full((1, _SIZE)),
            full((_SIZE, _SIZE)),
        ],
        out_specs=pl.BlockSpec((_BLOCK_B, 2 * _SIZE), lambda i: (i, 0)),
        out_shape=jax.ShapeDtypeStruct((b, 2 * _SIZE), jnp.float32),
        scratch_shapes=[pltpu.VMEM((8, _SIZE), jnp.float32)],
        compiler_params=pltpu.CompilerParams(
            dimension_semantics=("arbitrary",),
        ),
    )(childrens, childrens, tracking, W_iou, b_iou.reshape(1, -1),
      W_iou_track, W_forget, b_forget.reshape(1, -1), W_forget_track)


def kernel(childrens, tracking, W_iou, b_iou, W_forget, b_forget,
           W_iou_track, W_forget_track):
    return _tree_lstm(childrens, tracking, W_iou, b_iou, W_forget, b_forget,
                      W_iou_track, W_forget_track)
